# Initial kernel scaffold; baseline (speedup 1.0000x reference)
import jax
import jax.numpy as jnp
from jax.experimental import pallas as pl


def kernel(gt_ab, q_ab):
    raise NotImplementedError("write your pallas kernel here")


# speedup vs baseline: 5.6450x; 5.6450x over previous
import functools
import math

import jax
import jax.numpy as jnp
from jax.experimental import pallas as pl

SIGMA_ = 5.0
K_ = 5
BINS_ = 313


def _ct2_tile_kernel(q_ref, pts_ref, out_ref):
    px = pts_ref[0, 0:1, :]
    py = pts_ref[0, 1:2, :]
    qx = q_ref[:, 0:1]
    qy = q_ref[:, 1:2]
    bf = jnp.bfloat16
    f32 = jnp.float32
    cross = (qx.astype(bf).astype(f32) * px.astype(bf).astype(f32)
             + qy.astype(bf).astype(f32) * py.astype(bf).astype(f32))
    q_sq = qx * qx + qy * qy
    p_sq = px * px + py * py
    d2 = jnp.maximum((q_sq + p_sq) - 2.0 * cross, 0.0)

    iota = jax.lax.broadcasted_iota(jnp.int32, d2.shape, 0)
    inf = jnp.float32(jnp.inf)
    scale = jnp.float32(-1.0 / (2.0 * SIGMA_ * SIGMA_))

    cur = d2
    idxs = []
    ws = []
    for k in range(K_):
        m = jnp.min(cur, axis=0, keepdims=True)
        cand = jnp.where(cur == m, iota, BINS_)
        idx = jnp.min(cand, axis=0, keepdims=True)
        idxs.append(idx)
        ws.append(jnp.exp(m * scale))
        if k + 1 < K_:
            cur = jnp.where(iota == idx, inf, cur)

    inv = 1.0 / (ws[0] + ws[1] + ws[2] + ws[3] + ws[4])
    out = jnp.zeros(d2.shape, jnp.float32)
    for k in range(K_):
        out = jnp.where(iota == idxs[k], ws[k] * inv, out)
    out_ref[0, :, :] = out


@jax.jit
def kernel(gt_ab, q_ab):
    bs, _, H, W = gt_ab.shape
    hw = H * W
    nb = 3584
    pts = gt_ab.reshape(bs, 2, hw)
    grid = (bs, hw // nb)
    out = pl.pallas_call(
        _ct2_tile_kernel,
        grid=grid,
        in_specs=[
            pl.BlockSpec((BINS_, 2), lambda i, j: (0, 0)),
            pl.BlockSpec((1, 2, nb), lambda i, j: (i, 0, j)),
        ],
        out_specs=pl.BlockSpec((1, BINS_, nb), lambda i, j: (i, 0, j)),
        out_shape=jax.ShapeDtypeStruct((bs, BINS_, hw), jnp.float32),
    )(q_ab, pts)
    return out.reshape(bs, BINS_, H, W)


# speedup vs baseline: 31.5834x; 5.5949x over previous
import jax
import jax.numpy as jnp
from jax.experimental import pallas as pl

SIGMA_ = 5.0
K_ = 5
BINS_ = 313
IDX_MASK_ = (1 << 9) - 1
INT_MAX_ = jnp.iinfo(jnp.int32).max


def _ct2_tile_kernel(qx_ref, qy_ref, qsq_ref, qi_ref, pts_ref, out_ref):
    bf = jnp.bfloat16
    f32 = jnp.float32
    px = pts_ref[0, 0]
    py = pts_ref[0, 1]
    pxb = px.astype(bf).astype(f32)
    pyb = py.astype(bf).astype(f32)
    p_sq = px * px + py * py

    qxb = qx_ref[...].astype(f32)
    qyb = qy_ref[...].astype(f32)
    q_sq = qsq_ref[...]
    qi = qi_ref[...]

    cross = qxb * pxb[None] + qyb * pyb[None]
    d2 = jnp.maximum((q_sq + p_sq[None]) - 2.0 * cross, 0.0)

    inf = jnp.float32(jnp.inf)
    scale = jnp.float32(-1.0 / (2.0 * SIGMA_ * SIGMA_))
    cur = d2
    idxs = []
    ws = []
    for k in range(K_):
        m = jnp.min(cur, axis=0)
        cand = jnp.where(cur == m[None], qi, BINS_)
        idx = jnp.min(cand, axis=0)
        idxs.append(idx)
        ws.append(jnp.exp(m * scale))
        if k + 1 < K_:
            cur = jnp.where(qi == idx[None], inf, cur)

    inv = 1.0 / (ws[0] + ws[1] + ws[2] + ws[3] + ws[4])

    acc = jnp.zeros(d2.shape, f32)
    for k in range(K_):
        acc = jnp.where(qi == idxs[k][None], (ws[k] * inv)[None], acc)
    out_ref[0] = acc


@jax.jit
def kernel(gt_ab, q_ab):
    bs, _, H, W = gt_ab.shape
    hb = 16
    bf = jnp.bfloat16
    f32 = jnp.float32
    qxb = q_ab[:, 0].astype(bf).reshape(BINS_, 1, 1)
    qyb = q_ab[:, 1].astype(bf).reshape(BINS_, 1, 1)
    qsq = (q_ab[:, 0] * q_ab[:, 0] + q_ab[:, 1] * q_ab[:, 1]).reshape(BINS_, 1, 1)
    qi = jnp.arange(BINS_, dtype=jnp.int32).reshape(BINS_, 1, 1)
    grid = (bs, H // hb)
    qspec = pl.BlockSpec((BINS_, 1, 1), lambda i, j: (0, 0, 0))
    return pl.pallas_call(
        _ct2_tile_kernel,
        grid=grid,
        in_specs=[
            qspec, qspec, qspec, qspec,
            pl.BlockSpec((1, 2, hb, W), lambda i, j: (i, 0, j, 0)),
        ],
        out_specs=pl.BlockSpec((1, BINS_, hb, W), lambda i, j: (i, 0, j, 0)),
        out_shape=jax.ShapeDtypeStruct((bs, BINS_, H, W), jnp.float32),
    )(qxb, qyb, qsq, qi, gt_ab)


# speedup vs baseline: 43.4532x; 1.3758x over previous
import jax
import jax.numpy as jnp
from jax.experimental import pallas as pl

SIGMA_ = 5.0
K_ = 5
BINS_ = 313


IDX_MASK_ = (1 << 9) - 1
INT_MAX_ = jnp.iinfo(jnp.int32).max


def _ct2_tile_kernel(qx_ref, qy_ref, qsq_ref, qi_ref, pts_ref, out_ref):
    bf = jnp.bfloat16
    f32 = jnp.float32
    px = pts_ref[0, 0]
    py = pts_ref[0, 1]
    pxb = px.astype(bf).astype(f32)
    pyb = py.astype(bf).astype(f32)
    p_sq = px * px + py * py

    qxb = qx_ref[...].astype(f32)
    qyb = qy_ref[...].astype(f32)
    q_sq = qsq_ref[...]
    qi = qi_ref[...]

    cross = qxb * pxb[None] + qyb * pyb[None]
    d2 = jnp.maximum((q_sq + p_sq[None]) - 2.0 * cross, 0.0)
    key0 = (jax.lax.bitcast_convert_type(d2, jnp.int32) & ~IDX_MASK_) | qi

    cur = key0
    kms = []
    for k in range(K_):
        km = jnp.min(cur, axis=0)
        kms.append(km)
        if k + 1 < K_:
            cur = jnp.where(cur == km[None], INT_MAX_, cur)

    scale = jnp.float32(-1.0 / (2.0 * SIGMA_ * SIGMA_))
    ws = [jnp.exp(jax.lax.bitcast_convert_type(km & ~IDX_MASK_, f32) * scale)
          for km in kms]
    inv = 1.0 / (ws[0] + ws[1] + ws[2] + ws[3] + ws[4])

    dq = jax.lax.bitcast_convert_type(key0 & ~IDX_MASK_, f32)
    e = jnp.exp(dq * scale) * inv[None]
    out_ref[0] = jnp.where(key0 <= kms[K_ - 1][None], e, 0.0)


@jax.jit
def kernel(gt_ab, q_ab):
    bs, _, H, W = gt_ab.shape
    hb = 16
    bf = jnp.bfloat16
    qxb = q_ab[:, 0].astype(bf).reshape(BINS_, 1, 1)
    qyb = q_ab[:, 1].astype(bf).reshape(BINS_, 1, 1)
    qsq = (q_ab[:, 0] * q_ab[:, 0] + q_ab[:, 1] * q_ab[:, 1]).reshape(BINS_, 1, 1)
    qi = jnp.arange(BINS_, dtype=jnp.int32).reshape(BINS_, 1, 1)
    grid = (bs, H // hb)
    qspec = pl.BlockSpec((BINS_, 1, 1), lambda i, j: (0, 0, 0))
    return pl.pallas_call(
        _ct2_tile_kernel,
        grid=grid,
        in_specs=[
            qspec, qspec, qspec, qspec,
            pl.BlockSpec((1, 2, hb, W), lambda i, j: (i, 0, j, 0)),
        ],
        out_specs=pl.BlockSpec((1, BINS_, hb, W), lambda i, j: (i, 0, j, 0)),
        out_shape=jax.ShapeDtypeStruct((bs, BINS_, H, W), jnp.float32),
    )(qxb, qyb, qsq, qi, gt_ab)


# speedup vs baseline: 47.8218x; 1.1005x over previous
import jax
import jax.numpy as jnp
from jax.experimental import pallas as pl

SIGMA_ = 5.0
K_ = 5
BINS_ = 313


IDX_MASK_ = (1 << 9) - 1
INT_MAX_ = jnp.iinfo(jnp.int32).max


def _ct2_tile_kernel(qx_ref, qy_ref, qsq_ref, qi_ref, pts_ref, out_ref):
    bf = jnp.bfloat16
    f32 = jnp.float32
    px = pts_ref[0, 0]
    py = pts_ref[0, 1]
    pxb = px.astype(bf).astype(f32)
    pyb = py.astype(bf).astype(f32)
    p_sq = px * px + py * py

    qxb = qx_ref[...].astype(f32)
    qyb = qy_ref[...].astype(f32)
    q_sq = qsq_ref[...]
    qi = qi_ref[...]

    cross = qxb * pxb[None] + qyb * pyb[None]
    d2 = jnp.maximum((q_sq + p_sq[None]) - 2.0 * cross, 0.0)
    key0 = (jax.lax.bitcast_convert_type(d2, jnp.int32) & ~IDX_MASK_) | qi

    cur = key0
    kms = []
    for k in range(K_):
        km = jnp.min(cur, axis=0)
        kms.append(km)
        if k + 1 < K_:
            cur = jnp.where(cur == km[None], INT_MAX_, cur)

    scale = jnp.float32(-1.0 / (2.0 * SIGMA_ * SIGMA_))
    ws = [jnp.exp(jax.lax.bitcast_convert_type(km & ~IDX_MASK_, f32) * scale)
          for km in kms]
    nlogs = -jnp.log(ws[0] + ws[1] + ws[2] + ws[3] + ws[4])

    e = jnp.exp(d2 * scale + nlogs[None])
    out_ref[0] = jnp.where(key0 <= kms[K_ - 1][None], e, 0.0)


@jax.jit
def kernel(gt_ab, q_ab):
    bs, _, H, W = gt_ab.shape
    hb = 16
    bf = jnp.bfloat16
    qxb = q_ab[:, 0].astype(bf).reshape(BINS_, 1, 1)
    qyb = q_ab[:, 1].astype(bf).reshape(BINS_, 1, 1)
    qsq = (q_ab[:, 0] * q_ab[:, 0] + q_ab[:, 1] * q_ab[:, 1]).reshape(BINS_, 1, 1)
    qi = jnp.arange(BINS_, dtype=jnp.int32).reshape(BINS_, 1, 1)
    grid = (bs, H // hb)
    qspec = pl.BlockSpec((BINS_, 1, 1), lambda i, j: (0, 0, 0))
    return pl.pallas_call(
        _ct2_tile_kernel,
        grid=grid,
        in_specs=[
            qspec, qspec, qspec, qspec,
            pl.BlockSpec((1, 2, hb, W), lambda i, j: (i, 0, j, 0)),
        ],
        out_specs=pl.BlockSpec((1, BINS_, hb, W), lambda i, j: (i, 0, j, 0)),
        out_shape=jax.ShapeDtypeStruct((bs, BINS_, H, W), jnp.float32),
    )(qxb, qyb, qsq, qi, gt_ab)
